# baseline (speedup 1.0000x reference)
import functools

import jax
import jax.numpy as jnp
from jax import lax
from jax.experimental import pallas as pl
from jax.experimental.pallas import tpu as pltpu
from jax.experimental.pallas import tpu_sc as plsc

NUM_EMB = 1000000
EMBEDDING_DIM = 32


@functools.partial(jax.jit, static_argnums=(0, 1))
def _gather_call(N, K, idx_flat, table):
    D = EMBEDDING_DIM
    info = plsc.get_sparse_core_info()
    NC, NS = info.num_cores, info.num_subcores
    NW = NC * NS
    span = N // NW
    nchunks = span // K
    assert span % K == 0 and N % NW == 0
    mesh = plsc.VectorSubcoreMesh(core_axis_name="c", subcore_axis_name="s")

    @functools.partial(
        pl.kernel,
        mesh=mesh,
        out_type=jax.ShapeDtypeStruct((N, D), jnp.float32),
        scratch_types=[
            pltpu.VMEM((2, K), jnp.int32),
            pltpu.VMEM((2, K, D), jnp.float32),
            pltpu.SemaphoreType.DMA,
            pltpu.SemaphoreType.DMA,
        ],
        compiler_params=pltpu.CompilerParams(
            use_tc_tiling_on_sc=False, needs_layout_passes=False
        ),
    )
    def k(idx_hbm, table_hbm, out_hbm, idx_v, blk_v, gsem0, gsem1):
        gsems = (gsem0, gsem1)
        wid = lax.axis_index("s") * NC + lax.axis_index("c")
        base = wid * span

        def start(c, b):
            pltpu.sync_copy(idx_hbm.at[pl.ds(base + c * K, K)], idx_v.at[b])
            pltpu.async_copy(table_hbm.at[idx_v.at[b]], blk_v.at[b], gsems[b])

        start(0, 0)
        start(1, 1)

        def body(n, carry):
            for b in range(2):
                c = n * 2 + b
                pltpu.make_async_copy(
                    table_hbm.at[idx_v.at[b]], blk_v.at[b], gsems[b]
                ).wait()
                pltpu.sync_copy(blk_v.at[b], out_hbm.at[pl.ds(base + c * K, K)])

                @pl.when(c + 2 < nchunks)
                def _():
                    start(c + 2, b)

            return carry

        lax.fori_loop(0, nchunks // 2, body, 0)

    return k(idx_flat, table)


def kernel(input, table):
    I, J = input.shape
    idx_flat = input.reshape(-1).astype(jnp.int32)
    out_flat = _gather_call(I * J, 1600, idx_flat, table)
    return out_flat.reshape(I, J, EMBEDDING_DIM)
